# baseline (speedup 1.0000x reference)
import functools

import jax
import jax.numpy as jnp
from jax import lax
from jax.experimental import pallas as pl
from jax.experimental.pallas import tpu as pltpu
from jax.experimental.pallas import tpu_sc as plsc

NUM_CAT = 1000
EMBED_DIM = 64
RARE_THRESHOLD = 0.01
NUM_RARE = 500
RARE_PAD = 512

NC = 2
NS = 16
NW = NC * NS


def _prep_body(freqs_ref, w_ref, rare_ref, out_ref):
    f = freqs_ref[...]
    mask = f < RARE_THRESHOLD
    s = lax.rsqrt(f + 1e-8)
    s = s / (jnp.sum(s) / NUM_CAT)
    maskf = mask.astype(jnp.float32)
    ii = lax.broadcasted_iota(jnp.int32, (NUM_CAT, NUM_CAT), 0)
    jj = lax.broadcasted_iota(jnp.int32, (NUM_CAT, NUM_CAT), 1)
    strict_lower = (jj < ii).astype(jnp.float32)
    rank = jnp.dot(strict_lower, maskf, preferred_element_type=jnp.float32)
    rank_i = jnp.clip(rank.astype(jnp.int32), 0, NUM_RARE - 1)
    rr = lax.broadcasted_iota(jnp.int32, (NUM_CAT, RARE_PAD), 1)
    onehot = ((rank_i == rr) & mask).astype(jnp.float32)
    rare_rows = jnp.dot(onehot, rare_ref[...], preferred_element_type=jnp.float32)
    out_ref[...] = jnp.where(mask, rare_rows, w_ref[...]) * s


def _prep_table(freqs_col, w, rare_pad, interpret=False):
    return pl.pallas_call(
        _prep_body,
        out_shape=jax.ShapeDtypeStruct((NUM_CAT, EMBED_DIM), jnp.float32),
        interpret=interpret,
    )(freqs_col, w, rare_pad)


def _make_sc_gather(n_chunks, chunk, n_half):
    h_per_w = n_chunks * chunk
    mesh = plsc.VectorSubcoreMesh(core_axis_name="c", subcore_axis_name="s")

    @functools.partial(
        pl.kernel,
        mesh=mesh,
        out_type=(
            jax.ShapeDtypeStruct((n_half, EMBED_DIM), jnp.float32),
            jax.ShapeDtypeStruct((n_half, EMBED_DIM), jnp.float32),
        ),
        scratch_types=[
            pltpu.VMEM((chunk,), jnp.int32),
            pltpu.VMEM((chunk,), jnp.int32),
            pltpu.VMEM((2, chunk, EMBED_DIM), jnp.float32),
            pltpu.SemaphoreType.DMA,
            pltpu.SemaphoreType.DMA,
            pltpu.SemaphoreType.DMA,
            pltpu.SemaphoreType.DMA,
        ],
        compiler_params=pltpu.CompilerParams(use_tc_tiling_on_sc=False),
    )
    def gather_k(table_hbm, idx_e_hbm, idx_o_hbm, out_e, out_o, idx_ev, idx_ov,
                 rows_v, ge, go, se, so):
        wid = lax.axis_index("s") * NC + lax.axis_index("c")
        base = wid * h_per_w

        def body(k, carry):
            off = pl.multiple_of(base + k * chunk, 8)
            pltpu.sync_copy(idx_e_hbm.at[pl.ds(off, chunk)], idx_ev)
            de = pltpu.async_copy(table_hbm.at[idx_ev], rows_v.at[0], ge)
            pltpu.sync_copy(idx_o_hbm.at[pl.ds(off, chunk)], idx_ov)
            do = pltpu.async_copy(table_hbm.at[idx_ov], rows_v.at[1], go)
            de.wait()
            dse = pltpu.async_copy(rows_v.at[0], out_e.at[pl.ds(off, chunk)], se)
            do.wait()
            dso = pltpu.async_copy(rows_v.at[1], out_o.at[pl.ds(off, chunk)], so)
            dse.wait()
            dso.wait()
            return carry

        lax.fori_loop(0, n_chunks, body, 0)

    return gather_k


_N_CHUNKS = 10
_CHUNK = 320
_SC_GATHER_CACHE = {}


def _sc_gather(n_half):
    key = (_N_CHUNKS, _CHUNK, n_half)
    if key not in _SC_GATHER_CACHE:
        _SC_GATHER_CACHE[key] = _make_sc_gather(*key)
    return _SC_GATHER_CACHE[key]


def kernel(category_ids, W, rare_W, freqs):
    batch, hist = category_ids.shape
    n_half = batch * hist // 2
    freqs_col = freqs.reshape(NUM_CAT, 1)
    rare_pad = jnp.pad(rare_W, ((0, RARE_PAD - NUM_RARE), (0, 0)))
    eff = _prep_table(freqs_col, W, rare_pad)
    idx2 = category_ids.reshape(n_half, 2).astype(jnp.int32)
    out_e, out_o = _sc_gather(n_half)(eff, idx2[:, 0], idx2[:, 1])
    out = jnp.concatenate([out_e[:, None, :], out_o[:, None, :]], axis=1)
    return out.reshape(batch, hist, EMBED_DIM)


# speedup vs baseline: 1.5144x; 1.5144x over previous
import functools

import jax
import jax.numpy as jnp
from jax import lax
from jax.experimental import pallas as pl
from jax.experimental.pallas import tpu as pltpu
from jax.experimental.pallas import tpu_sc as plsc

NUM_CAT = 1000
EMBED_DIM = 64
RARE_THRESHOLD = 0.01
NUM_RARE = 500
RARE_PAD = 512

NC = 2
NS = 16
NW = NC * NS


def _prep_body(freqs_ref, w_ref, rare_ref, out_ref):
    f = freqs_ref[...]
    mask = f < RARE_THRESHOLD
    s = lax.rsqrt(f + 1e-8)
    s = s / (jnp.sum(s) / NUM_CAT)
    maskf = mask.astype(jnp.float32)
    ii = lax.broadcasted_iota(jnp.int32, (NUM_CAT, NUM_CAT), 0)
    jj = lax.broadcasted_iota(jnp.int32, (NUM_CAT, NUM_CAT), 1)
    strict_lower = (jj < ii).astype(jnp.float32)
    rank = jnp.dot(strict_lower, maskf, preferred_element_type=jnp.float32)
    rank_i = jnp.clip(rank.astype(jnp.int32), 0, NUM_RARE - 1)
    rr = lax.broadcasted_iota(jnp.int32, (NUM_CAT, RARE_PAD), 1)
    onehot = ((rank_i == rr) & mask).astype(jnp.float32)
    rare_rows = jnp.dot(onehot, rare_ref[...], preferred_element_type=jnp.float32)
    eff = jnp.where(mask, rare_rows, w_ref[...]) * s
    out_ref[...] = eff.astype(jnp.bfloat16)


def _prep_table(freqs_col, w, rare_pad, interpret=False):
    return pl.pallas_call(
        _prep_body,
        out_shape=jax.ShapeDtypeStruct((NUM_CAT, EMBED_DIM), jnp.bfloat16),
        interpret=interpret,
    )(freqs_col, w, rare_pad)


def _make_sc_gather(n_chunks, chunk, n_total):
    b_per_w = n_chunks * chunk
    mesh = plsc.VectorSubcoreMesh(core_axis_name="c", subcore_axis_name="s")

    @functools.partial(
        pl.kernel,
        mesh=mesh,
        out_type=jax.ShapeDtypeStruct((n_total, EMBED_DIM), jnp.bfloat16),
        scratch_types=[
            pltpu.VMEM((chunk,), jnp.int32),
            pltpu.VMEM((chunk,), jnp.int32),
            pltpu.VMEM((2, chunk, EMBED_DIM), jnp.bfloat16),
            pltpu.SemaphoreType.DMA,
            pltpu.SemaphoreType.DMA,
            pltpu.SemaphoreType.DMA,
            pltpu.SemaphoreType.DMA,
        ],
        compiler_params=pltpu.CompilerParams(use_tc_tiling_on_sc=False),
    )
    def gather_k(table_hbm, idx_hbm, out_hbm, idx_a, idx_b, rows_v, g0, g1, s0, s1):
        gsem = (g0, g1)
        ssem = (s0, s1)
        idx_v = (idx_a, idx_b)
        wid = lax.axis_index("s") * NC + lax.axis_index("c")
        base_i = wid * b_per_w

        def idx_load(k):
            pltpu.sync_copy(idx_hbm.at[pl.ds(base_i + k * chunk, chunk)], idx_v[k % 2])

        def gather_start(k):
            return pltpu.async_copy(table_hbm.at[idx_v[k % 2]], rows_v.at[k % 2], gsem[k % 2])

        def store_start(k):
            return pltpu.async_copy(
                rows_v.at[k % 2],
                out_hbm.at[pl.ds(base_i + k * chunk, chunk)],
                ssem[k % 2],
            )

        gd = [None] * n_chunks
        sd = [None] * n_chunks
        idx_load(0)
        gd[0] = gather_start(0)
        for k in range(n_chunks):
            if k + 1 < n_chunks:
                idx_load(k + 1)
            gd[k].wait()
            if k + 1 < n_chunks:
                if k >= 1:
                    sd[k - 1].wait()
                gd[k + 1] = gather_start(k + 1)
            sd[k] = store_start(k)
        if n_chunks >= 2:
            sd[n_chunks - 2].wait()
        sd[n_chunks - 1].wait()

    return gather_k


_N_CHUNKS = 8
_CHUNK = 800
_SC_GATHER_CACHE = {}


def _sc_gather(n_total):
    key = (_N_CHUNKS, _CHUNK, n_total)
    if key not in _SC_GATHER_CACHE:
        _SC_GATHER_CACHE[key] = _make_sc_gather(*key)
    return _SC_GATHER_CACHE[key]


def kernel(category_ids, W, rare_W, freqs):
    batch, hist = category_ids.shape
    freqs_col = freqs.reshape(NUM_CAT, 1)
    rare_pad = jnp.pad(rare_W, ((0, RARE_PAD - NUM_RARE), (0, 0)))
    eff = _prep_table(freqs_col, W, rare_pad)
    idx_flat = category_ids.reshape(-1).astype(jnp.int32)
    out = _sc_gather(batch * hist)(eff, idx_flat)
    return out.astype(jnp.float32).reshape(batch, hist, EMBED_DIM)


# speedup vs baseline: 1.7298x; 1.1422x over previous
import functools

import jax
import jax.numpy as jnp
from jax import lax
from jax.experimental import pallas as pl
from jax.experimental.pallas import tpu as pltpu
from jax.experimental.pallas import tpu_sc as plsc

NUM_CAT = 1000
EMBED_DIM = 64
RARE_THRESHOLD = 0.01
NUM_RARE = 500
RARE_PAD = 512

NC = 2
NS = 16
NW = NC * NS


def _prep_body(freqs_ref, w_ref, rare_ref, out_ref):
    f = freqs_ref[...]
    mask = f < RARE_THRESHOLD
    s = lax.rsqrt(f + 1e-8)
    s = s / (jnp.sum(s) / NUM_CAT)
    maskf = mask.astype(jnp.float32)
    ii = lax.broadcasted_iota(jnp.int32, (NUM_CAT, NUM_CAT), 0)
    jj = lax.broadcasted_iota(jnp.int32, (NUM_CAT, NUM_CAT), 1)
    strict_lower = (jj < ii).astype(jnp.float32)
    rank = jnp.dot(strict_lower, maskf, preferred_element_type=jnp.float32)
    rank_i = jnp.clip(rank.astype(jnp.int32), 0, NUM_RARE - 1)
    rr = lax.broadcasted_iota(jnp.int32, (NUM_CAT, RARE_PAD), 1)
    onehot = ((rank_i == rr) & mask).astype(jnp.float32)
    rare_rows = jnp.dot(onehot, rare_ref[...], preferred_element_type=jnp.float32)
    out_ref[...] = jnp.where(mask, rare_rows, w_ref[...]) * s


def _prep_table(freqs_col, w, rare_pad, interpret=False):
    return pl.pallas_call(
        _prep_body,
        out_shape=jax.ShapeDtypeStruct((NUM_CAT, EMBED_DIM), jnp.float32),
        interpret=interpret,
    )(freqs_col, w, rare_pad)


def _relayout_body(in_ref, out_ref):
    x = in_ref[...]
    e = x[:, :EMBED_DIM]
    o = x[:, EMBED_DIM:]
    z = jnp.concatenate([e[:, None, :], o[:, None, :]], axis=1)
    out_ref[...] = z.reshape(out_ref.shape)


def _relayout(flat, batch, hist, grid):
    bb = batch // grid
    rows2 = bb * hist // 2
    pairs = batch * hist // 2
    x2 = flat.reshape(pairs, 2 * EMBED_DIM)
    return pl.pallas_call(
        _relayout_body,
        grid=(grid,),
        in_specs=[pl.BlockSpec((rows2, 2 * EMBED_DIM), lambda g: (g, 0))],
        out_specs=pl.BlockSpec((bb, hist, EMBED_DIM), lambda g: (g, 0, 0)),
        out_shape=jax.ShapeDtypeStruct((batch, hist, EMBED_DIM), jnp.float32),
    )(x2)


def _make_sc_gather(n_chunks, chunk_b, batch, hist):
    b_per_w = n_chunks * chunk_b
    chunk = chunk_b * hist
    mesh = plsc.VectorSubcoreMesh(core_axis_name="c", subcore_axis_name="s")

    @functools.partial(
        pl.kernel,
        mesh=mesh,
        out_type=jax.ShapeDtypeStruct((batch * hist, EMBED_DIM), jnp.float32),
        scratch_types=[
            pltpu.VMEM((chunk,), jnp.int32),
            pltpu.VMEM((chunk,), jnp.int32),
            pltpu.VMEM((2, chunk, EMBED_DIM), jnp.float32),
            pltpu.SemaphoreType.DMA,
            pltpu.SemaphoreType.DMA,
            pltpu.SemaphoreType.DMA,
            pltpu.SemaphoreType.DMA,
        ],
        compiler_params=pltpu.CompilerParams(use_tc_tiling_on_sc=False),
    )
    def gather_k(table_hbm, idx_hbm, out_hbm, idx_a, idx_b, rows_v, g0, g1, s0, s1):
        gsem = (g0, g1)
        ssem = (s0, s1)
        idx_v = (idx_a, idx_b)
        wid = lax.axis_index("s") * NC + lax.axis_index("c")
        base_b = wid * b_per_w
        base_i = base_b * hist

        def idx_load(k):
            pltpu.sync_copy(idx_hbm.at[pl.ds(base_i + k * chunk, chunk)], idx_v[k % 2])

        def gather_start(k):
            return pltpu.async_copy(table_hbm.at[idx_v[k % 2]], rows_v.at[k % 2], gsem[k % 2])

        def stores_start(k):
            return [
                pltpu.async_copy(
                    rows_v.at[k % 2],
                    out_hbm.at[pl.ds(base_i + k * chunk, chunk)],
                    ssem[k % 2],
                )
            ]

        gd = [None] * n_chunks
        sd = [None] * n_chunks
        idx_load(0)
        gd[0] = gather_start(0)
        for k in range(n_chunks):
            if k + 1 < n_chunks:
                idx_load(k + 1)
            gd[k].wait()
            if k + 1 < n_chunks:
                if k >= 1:
                    for d in sd[k - 1]:
                        d.wait()
                gd[k + 1] = gather_start(k + 1)
            sd[k] = stores_start(k)
        if n_chunks >= 2:
            for d in sd[n_chunks - 2]:
                d.wait()
        for d in sd[n_chunks - 1]:
            d.wait()

    return gather_k


_N_CHUNKS = 8
_CHUNK_B = 16
_SC_GATHER_CACHE = {}


def _sc_gather(batch, hist):
    key = (_N_CHUNKS, _CHUNK_B, batch, hist)
    if key not in _SC_GATHER_CACHE:
        _SC_GATHER_CACHE[key] = _make_sc_gather(*key)
    return _SC_GATHER_CACHE[key]


def kernel(category_ids, W, rare_W, freqs):
    batch, hist = category_ids.shape
    freqs_col = freqs.reshape(NUM_CAT, 1)
    rare_pad = jnp.pad(rare_W, ((0, RARE_PAD - NUM_RARE), (0, 0)))
    eff = _prep_table(freqs_col, W, rare_pad)
    idx_flat = category_ids.reshape(-1).astype(jnp.int32)
    out2d = _sc_gather(batch, hist)(eff, idx_flat)
    return _relayout(out2d.reshape(-1), batch, hist, grid=32)


# speedup vs baseline: 2.2248x; 1.2862x over previous
import functools

import jax
import jax.numpy as jnp
from jax import lax
from jax.experimental import pallas as pl
from jax.experimental.pallas import tpu as pltpu
from jax.experimental.pallas import tpu_sc as plsc

NUM_CAT = 1000
EMBED_DIM = 64
RARE_THRESHOLD = 0.01
NUM_RARE = 500
RARE_PAD = 512

NC = 2
NS = 16
NW = NC * NS


def _prep_body(freqs_ref, w_ref, rare_ref, out_ref):
    f = freqs_ref[...]
    mask = f < RARE_THRESHOLD
    s = lax.rsqrt(f + 1e-8)
    s = s / (jnp.sum(s) / NUM_CAT)
    maskf = mask.astype(jnp.float32)
    ii = lax.broadcasted_iota(jnp.int32, (NUM_CAT, NUM_CAT), 0)
    jj = lax.broadcasted_iota(jnp.int32, (NUM_CAT, NUM_CAT), 1)
    strict_lower = (jj < ii).astype(jnp.float32)
    rank = jnp.dot(strict_lower, maskf, preferred_element_type=jnp.float32)
    rank_i = jnp.clip(rank.astype(jnp.int32), 0, NUM_RARE - 1)
    rr = lax.broadcasted_iota(jnp.int32, (NUM_CAT, RARE_PAD), 1)
    onehot = ((rank_i == rr) & mask).astype(jnp.float32)
    rare_rows = jnp.dot(onehot, rare_ref[...], preferred_element_type=jnp.float32)
    out_ref[...] = jnp.where(mask, rare_rows, w_ref[...]) * s


def _prep_table(freqs_col, w, rare_pad, interpret=False):
    return pl.pallas_call(
        _prep_body,
        out_shape=jax.ShapeDtypeStruct((NUM_CAT, EMBED_DIM), jnp.float32),
        interpret=interpret,
    )(freqs_col, w, rare_pad)


def _make_sc_gather(n_chunks, chunk_b, batch, hist):
    b_per_w = n_chunks * chunk_b
    chunk = chunk_b * hist
    mesh = plsc.VectorSubcoreMesh(core_axis_name="c", subcore_axis_name="s")

    @functools.partial(
        pl.kernel,
        mesh=mesh,
        out_type=jax.ShapeDtypeStruct((batch, hist, EMBED_DIM), jnp.float32),
        scratch_types=[
            pltpu.VMEM((chunk,), jnp.int32),
            pltpu.VMEM((chunk,), jnp.int32),
            pltpu.VMEM((2, chunk, EMBED_DIM), jnp.float32),
            pltpu.SemaphoreType.DMA,
            pltpu.SemaphoreType.DMA,
            pltpu.SemaphoreType.DMA,
            pltpu.SemaphoreType.DMA,
        ],
        compiler_params=pltpu.CompilerParams(use_tc_tiling_on_sc=False),
    )
    def gather_k(table_hbm, idx_hbm, out_hbm, idx_a, idx_b, rows_v, g0, g1, s0, s1):
        gsem = (g0, g1)
        ssem = (s0, s1)
        idx_v = (idx_a, idx_b)
        wid = lax.axis_index("s") * NC + lax.axis_index("c")
        base_b = wid * b_per_w
        base_i = base_b * hist

        def idx_load(k):
            pltpu.sync_copy(idx_hbm.at[pl.ds(base_i + k * chunk, chunk)], idx_v[k % 2])

        def gather_start(k):
            return pltpu.async_copy(table_hbm.at[idx_v[k % 2]], rows_v.at[k % 2], gsem[k % 2])

        def stores_start(k):
            buf = rows_v.at[k % 2]
            return [
                pltpu.async_copy(
                    buf.at[pl.ds(j * hist, hist)],
                    out_hbm.at[base_b + k * chunk_b + j],
                    ssem[k % 2],
                )
                for j in range(chunk_b)
            ]

        gd = [None] * n_chunks
        sd = [None] * n_chunks
        idx_load(0)
        gd[0] = gather_start(0)
        for k in range(n_chunks):
            if k + 1 < n_chunks:
                idx_load(k + 1)
            gd[k].wait()
            if k + 1 < n_chunks:
                if k >= 1:
                    for d in sd[k - 1]:
                        d.wait()
                gd[k + 1] = gather_start(k + 1)
            sd[k] = stores_start(k)
        if n_chunks >= 2:
            for d in sd[n_chunks - 2]:
                d.wait()
        for d in sd[n_chunks - 1]:
            d.wait()

    return gather_k


_N_CHUNKS = 8
_CHUNK_B = 16
_SC_GATHER_CACHE = {}


def _sc_gather(batch, hist):
    key = (_N_CHUNKS, _CHUNK_B, batch, hist)
    if key not in _SC_GATHER_CACHE:
        _SC_GATHER_CACHE[key] = _make_sc_gather(*key)
    return _SC_GATHER_CACHE[key]


def kernel(category_ids, W, rare_W, freqs):
    batch, hist = category_ids.shape
    freqs_col = freqs.reshape(NUM_CAT, 1)
    rare_pad = jnp.pad(rare_W, ((0, RARE_PAD - NUM_RARE), (0, 0)))
    eff = _prep_table(freqs_col, W, rare_pad)
    idx_flat = category_ids.reshape(-1).astype(jnp.int32)
    return _sc_gather(batch, hist)(eff, idx_flat)
